# baseline (speedup 1.0000x reference)
import jax
import jax.numpy as jnp
from jax.experimental import pallas as pl
from jax.experimental.pallas import tpu as pltpu

BATCH = 128
R = 64
NUM_ADAPTERS = 64
D_OUT = 4096


def _matmul_body(wids_ref, y_ref, b_ref, out_ref, acc_ref):
    a = pl.program_id(0)

    @pl.when(a == 0)
    def _():
        acc_ref[...] = jnp.zeros_like(acc_ref)

    mask = wids_ref[...] == a
    y_masked = jnp.where(mask, y_ref[...], jnp.zeros_like(y_ref))
    acc_ref[...] += jnp.dot(y_masked.astype(jnp.bfloat16), b_ref[0],
                            preferred_element_type=jnp.float32)

    @pl.when(a == NUM_ADAPTERS - 1)
    def _():
        out_ref[...] = (acc_ref[...] * 2.0).astype(out_ref.dtype)


def kernel(y, wids, lora_B):
    y2d = y.reshape(BATCH, R).astype(jnp.float32)
    wids2d = wids.reshape(BATCH, 1)

    out = pl.pallas_call(
        _matmul_body,
        grid=(NUM_ADAPTERS,),
        in_specs=[
            pl.BlockSpec((BATCH, 1), lambda a: (0, 0)),
            pl.BlockSpec((BATCH, R), lambda a: (0, 0)),
            pl.BlockSpec((1, R, D_OUT), lambda a: (a, 0, 0)),
        ],
        out_specs=pl.BlockSpec((BATCH, D_OUT), lambda a: (0, 0)),
        out_shape=jax.ShapeDtypeStruct((BATCH, D_OUT), jnp.float32),
        scratch_shapes=[pltpu.VMEM((BATCH, D_OUT), jnp.float32)],
        compiler_params=pltpu.CompilerParams(
            dimension_semantics=("arbitrary",),
        ),
    )(wids2d, y2d, lora_B.astype(jnp.bfloat16))
    return out.astype(y.dtype).reshape(BATCH, 1, D_OUT)
